# Initial kernel scaffold; baseline (speedup 1.0000x reference)
import jax
import jax.numpy as jnp
from jax.experimental import pallas as pl


def kernel(x, table):
    raise NotImplementedError("write your pallas kernel here")


# speedup vs baseline: 1.8794x; 1.8794x over previous
import functools

import jax
import jax.numpy as jnp
from jax import lax
from jax.experimental import pallas as pl
from jax.experimental.pallas import tpu as pltpu
from jax.experimental.pallas import tpu_sc as plsc

B = 16384
L = 50
D = 64
N = B * L
NC = 2
NS = 16
NW = NC * NS
PER_W = N // NW
G = 128
IDX_ROWS = PER_W // G
CH = 4
CHUNK = G * CH
N_CHUNK = PER_W // CHUNK


@functools.cache
def _build():
    mesh = plsc.VectorSubcoreMesh(
        core_axis_name="c", subcore_axis_name="s", num_cores=NC, num_subcores=NS
    )

    @functools.partial(
        pl.kernel,
        out_type=jax.ShapeDtypeStruct((N, D), jnp.float32),
        mesh=mesh,
        compiler_params=pltpu.CompilerParams(use_tc_tiling_on_sc=False),
        scratch_types=[
            pltpu.VMEM((IDX_ROWS, G), jnp.int32),
            pltpu.VMEM((CHUNK, D), jnp.float32),
            pltpu.VMEM((CHUNK, D), jnp.float32),
            pltpu.SemaphoreType.DMA,
            pltpu.SemaphoreType.DMA,
            pltpu.SemaphoreType.DMA,
            pltpu.SemaphoreType.DMA,
        ],
    )
    def _embed_gather(idx_hbm, table_hbm, out_hbm, idx_v, rows0, rows1,
                      gsem0, gsem1, ssem0, ssem1):
        wid = lax.axis_index("s") * NC + lax.axis_index("c")
        base = wid * PER_W
        pltpu.sync_copy(idx_hbm.at[pl.ds(wid * IDX_ROWS, IDX_ROWS)], idx_v)

        bufs = (rows0, rows1)
        gsems = (gsem0, gsem1)
        ssems = (ssem0, ssem1)

        def fire_gathers(c, b):
            r0 = c * CH
            for i in range(CH):
                pltpu.async_copy(
                    table_hbm.at[idx_v.at[r0 + i]],
                    bufs[b].at[pl.ds(i * G, G)],
                    gsems[b],
                )

        def drain_gathers(c, b):
            r0 = c * CH
            for i in range(CH):
                pltpu.make_async_copy(
                    table_hbm.at[idx_v.at[r0 + i]],
                    bufs[b].at[pl.ds(i * G, G)],
                    gsems[b],
                ).wait()

        def fire_store(c, b):
            pltpu.async_copy(
                bufs[b], out_hbm.at[pl.ds(base + c * CHUNK, CHUNK)], ssems[b]
            )

        def wait_store(c, b):
            pltpu.make_async_copy(
                bufs[b], out_hbm.at[pl.ds(base + c * CHUNK, CHUNK)], ssems[b]
            ).wait()

        fire_gathers(0, 0)
        fire_gathers(1, 1)
        drain_gathers(0, 0)
        fire_store(0, 0)
        drain_gathers(1, 1)
        fire_store(1, 1)

        @pl.loop(2, N_CHUNK, step=2)
        def _chunk(j):
            wait_store(j - 2, 0)
            fire_gathers(j, 0)
            wait_store(j - 1, 1)
            fire_gathers(j + 1, 1)
            drain_gathers(j, 0)
            fire_store(j, 0)
            drain_gathers(j + 1, 1)
            fire_store(j + 1, 1)

        wait_store(N_CHUNK - 2, 0)
        wait_store(N_CHUNK - 1, 1)

    return _embed_gather


def kernel(x, table):
    idx = x.reshape(N // G, G).astype(jnp.int32)
    out = _build()(idx, table)
    return out.reshape(B, L, D)
